# Initial kernel scaffold; baseline (speedup 1.0000x reference)
import jax
import jax.numpy as jnp
from jax.experimental import pallas as pl


def kernel(inputs, gating_kernel):
    raise NotImplementedError("write your pallas kernel here")


# speedup vs baseline: 1.7538x; 1.7538x over previous
import jax
import jax.numpy as jnp
from jax import lax
from jax.experimental import pallas as pl
from jax.experimental.pallas import tpu as pltpu
from jax.experimental.pallas import tpu_sc as plsc

K = 8192
B = 2
H = 512
W = 512
C = 96
GX = 256
GY = 256
N = GX * GY

NC = 2
NS = 16
L = 16
CHUNK = N // NS
NV = CHUNK // L
NBINS = 256
LEVELS = 4

ROWS_A = 64
ROWS_B = 32
HIGHEST = lax.Precision.HIGHEST


def _gate_body(x_ref, w_ref, g_ref):
    x = x_ref[0].astype(jnp.bfloat16).astype(jnp.float32)
    w = w_ref[...].astype(jnp.float32)
    xr = x.reshape(ROWS_A // 2, 2, W, C)
    s = (xr[:, 0] * w[0]).sum(-1) + (xr[:, 1] * w[1]).sum(-1)
    g_ref[0] = s.reshape(ROWS_A // 2, GY, 2).sum(-1)


def _apply_body(x_ref, g_ref, t_ref, p_ref, e_ref, o_ref):
    t = t_ref[pl.program_id(0), 0]
    g = g_ref[0]
    m = jnp.where(g >= t, g, 0.0)
    m2 = lax.dot(p_ref[...], m, precision=HIGHEST)
    m2 = lax.dot(m2, e_ref[...], precision=HIGHEST)
    o_ref[0] = x_ref[0] * m2[:, :, None]


def _select_body(g_hbm, out_hbm, u_ref, hist_ref, tot_ref,
                 all_ref, outv_ref, shared_ref):
    cid = lax.axis_index("c")
    sid = lax.axis_index("s")
    base = cid * N + sid * CHUNK

    lanes = lax.iota(jnp.int32, L)
    zeros16 = jnp.zeros((L,), jnp.int32)
    ones16 = jnp.ones((L,), jnp.int32)

    pltpu.sync_copy(g_hbm.at[pl.ds(base, CHUNK)], u_ref)

    def xform(i, carry):
        bits = u_ref[pl.ds(i * L, L)]
        u_ref[pl.ds(i * L, L)] = jnp.where(
            bits < 0, ~bits, bits | jnp.int32(-(2**31)))
        return carry
    lax.fori_loop(0, NV, xform, 0)

    prefix = jnp.int32(0)
    rank = jnp.int32(N - K + 1)

    for lvl in range(LEVELS):
        shift_cur = 24 - 8 * lvl
        shift_prev = 32 - 8 * lvl

        def zbody(i, carry):
            hist_ref[pl.ds(i * L, L)] = zeros16
            return carry
        lax.fori_loop(0, NBINS * L // L, zbody, 0)

        def hbody(i, carry, _shift_cur=shift_cur, _shift_prev=shift_prev,
                  _lvl=lvl, _prefix=prefix):
            u = u_ref[pl.ds(i * L, L)]
            digit = lax.shift_right_logical(u, _shift_cur) & 255
            idx = lanes * NBINS + digit
            if _lvl == 0:
                plsc.addupdate_scatter(hist_ref, [idx], ones16)
            else:
                match = lax.shift_right_logical(u, _shift_prev) == _prefix
                plsc.addupdate_scatter(hist_ref, [idx], ones16, mask=match)
            return carry
        lax.fori_loop(0, NV, hbody, 0)

        def tbody(chk, carry):
            acc = zeros16
            for lane in range(L):
                acc = acc + hist_ref[pl.ds(lane * NBINS + chk * L, L)]
            tot_ref[pl.ds(chk * L, L)] = acc
            return carry
        lax.fori_loop(0, NBINS // L, tbody, 0)

        pltpu.sync_copy(tot_ref, shared_ref.at[lvl, sid])
        plsc.subcore_barrier()
        pltpu.sync_copy(shared_ref.at[lvl], all_ref)

        def gbody(chk, carry):
            run, sel_bin, below = carry
            acc = zeros16
            for t in range(NS):
                acc = acc + all_ref[t, pl.ds(chk * L, L)]
            cum = plsc.cumsum(acc) + run
            run_next = jnp.max(cum)
            crossed = cum >= rank
            is_cross = jnp.logical_and(run < rank, run_next >= rank)
            lane_idx = jnp.sum(jnp.where(crossed, 0, 1))
            below_here = jnp.maximum(run, jnp.max(jnp.where(crossed, 0, cum)))
            sel_bin = jnp.where(is_cross, chk * L + lane_idx, sel_bin)
            below = jnp.where(is_cross, below_here, below)
            return (run_next, sel_bin, below)
        _, sel_bin, below = lax.fori_loop(
            0, NBINS // L, gbody,
            (jnp.int32(0), jnp.int32(0), jnp.int32(0)))

        prefix = jnp.bitwise_or(lax.shift_left(prefix, 8), sel_bin)
        rank = rank - below

    uvec = jnp.full((L,), prefix, dtype=jnp.int32)
    outv_ref[...] = jnp.where(uvec < 0, uvec & jnp.int32(0x7FFFFFFF), ~uvec)

    @pl.when(sid == 0)
    def _():
        pltpu.sync_copy(outv_ref, out_hbm.at[cid])


def _make_select():
    mesh = plsc.VectorSubcoreMesh(core_axis_name="c", subcore_axis_name="s")
    return pl.kernel(
        _select_body,
        out_type=jax.ShapeDtypeStruct((B, L), jnp.int32),
        mesh=mesh,
        scratch_types=[
            pltpu.VMEM((CHUNK,), jnp.int32),
            pltpu.VMEM((NBINS * L,), jnp.int32),
            pltpu.VMEM((NBINS,), jnp.int32),
            pltpu.VMEM((NS, NBINS), jnp.int32),
            pltpu.VMEM((L,), jnp.int32),
            pltpu.VMEM_SHARED((LEVELS, NS, NBINS), jnp.int32),
        ],
        compiler_params=pltpu.CompilerParams(needs_layout_passes=False),
    )


def _pass_a(inputs, w_tiled):
    return pl.pallas_call(
        _gate_body,
        grid=(B, H // ROWS_A),
        in_specs=[
            pl.BlockSpec((1, ROWS_A, W, C), lambda b, i: (b, i, 0, 0)),
            pl.BlockSpec((2, W, C), lambda b, i: (0, 0, 0)),
        ],
        out_specs=pl.BlockSpec((1, ROWS_A // 2, GY), lambda b, i: (b, i, 0)),
        out_shape=jax.ShapeDtypeStruct((B, GX, GY), jnp.float32),
        compiler_params=pltpu.CompilerParams(
            dimension_semantics=("parallel", "parallel")),
    )(inputs, w_tiled)


def _pass_b(inputs, g, thr, p_exp, e_exp):
    return pl.pallas_call(
        _apply_body,
        grid=(B, H // ROWS_B),
        in_specs=[
            pl.BlockSpec((1, ROWS_B, W, C), lambda b, i: (b, i, 0, 0)),
            pl.BlockSpec((1, ROWS_B // 2, GY), lambda b, i: (b, i, 0)),
            pl.BlockSpec(memory_space=pltpu.SMEM),
            pl.BlockSpec((ROWS_B, ROWS_B // 2), lambda b, i: (0, 0)),
            pl.BlockSpec((GY, W), lambda b, i: (0, 0)),
        ],
        out_specs=pl.BlockSpec((1, ROWS_B, W, C), lambda b, i: (b, i, 0, 0)),
        out_shape=jax.ShapeDtypeStruct((B, H, W, C), jnp.float32),
        compiler_params=pltpu.CompilerParams(
            dimension_semantics=("parallel", "parallel")),
    )(inputs, g, thr, p_exp, e_exp)


@jax.jit
def kernel(inputs, gating_kernel):
    gk = gating_kernel[..., 0].astype(jnp.bfloat16)
    w_tiled = jnp.tile(gk[:, None, :, :], (1, GY, 1, 1)).reshape(2, W, C)
    cols = jnp.arange(W)[:, None] // 2 == jnp.arange(GY)[None, :]
    p_exp = (jnp.arange(ROWS_B)[:, None] // 2 ==
             jnp.arange(ROWS_B // 2)[None, :]).astype(jnp.float32)
    e_exp = cols.T.astype(jnp.float32)

    g = _pass_a(inputs, w_tiled)
    gbits = lax.bitcast_convert_type(g, jnp.int32).reshape(B * N)
    thr = lax.bitcast_convert_type(_make_select()(gbits), jnp.float32)
    return _pass_b(inputs, g, thr, p_exp, e_exp)
